# baseline (speedup 1.0000x reference)
import functools

import jax
import jax.numpy as jnp
from jax import lax
from jax.experimental import pallas as pl
from jax.experimental.pallas import tpu as pltpu
from jax.experimental.pallas import tpu_sc as plsc

_NUM_CORES = 2
_NUM_SUBCORES = 16
_NUM_WORKERS = _NUM_CORES * _NUM_SUBCORES

_CHUNK = 8
_NBUF = 14
_B_PER_W = 1024
_N_CHUNKS = _B_PER_W // _CHUNK


def _make_gather(n_rows: int, d_model: int):
  mesh = plsc.VectorSubcoreMesh(core_axis_name="c", subcore_axis_name="s")

  @functools.partial(
      pl.kernel,
      mesh=mesh,
      out_type=jax.ShapeDtypeStruct((n_rows, d_model), jnp.float32),
      scratch_types=[
          pltpu.VMEM((_N_CHUNKS, _CHUNK), jnp.int32),
          pltpu.VMEM((_NBUF, _CHUNK, d_model), jnp.float32),
          [pltpu.SemaphoreType.DMA] * _NBUF,
          [pltpu.SemaphoreType.DMA] * _NBUF,
      ],
  )
  def gather_kernel(idx_hbm, table_hbm, out_hbm, idx_v, rows_v, gsems, wsems):
    wid = lax.axis_index("s") * _NUM_CORES + lax.axis_index("c")
    base = wid * _B_PER_W
    pltpu.sync_copy(idx_hbm.at[wid], idx_v)

    def g_copy(c, b):
      return pltpu.make_async_copy(
          table_hbm.at[idx_v.at[c]], rows_v.at[b], gsems[b])

    def w_copy(c, b):
      dst = out_hbm.at[pl.ds(base + c * _CHUNK, _CHUNK)]
      return pltpu.make_async_copy(rows_v.at[b], dst, wsems[b])

    for c in range(_NBUF - 1):
      g_copy(c, c).start()

    _MAIN = _N_CHUNKS - (_N_CHUNKS % _NBUF)

    @pl.loop(0, _MAIN, step=_NBUF)
    def _chunks(c0):
      for b in range(_NBUF):
        c = c0 + b
        g_copy(c, b).wait()
        w_copy(c, b).start()
        b2 = (b + _NBUF - 1) % _NBUF

        @pl.when(c > 0)
        def _():
          w_copy(c - 1, b2).wait()

        @pl.when(c + _NBUF - 1 < _N_CHUNKS)
        def _():
          g_copy(c + _NBUF - 1, b2).start()

    for c in range(_MAIN, _N_CHUNKS):
      b = c % _NBUF
      g_copy(c, b).wait()
      w_copy(c, b).start()
      w_copy(c - 1, (b + _NBUF - 1) % _NBUF).wait()

    w_copy(_N_CHUNKS - 1, (_N_CHUNKS - 1) % _NBUF).wait()

  return gather_kernel


def kernel(positions, table):
  b, t = positions.shape
  n = b * t
  idx = positions.reshape(_NUM_WORKERS, _N_CHUNKS, _CHUNK).astype(jnp.int32)
  out = _make_gather(n, table.shape[1])(idx, table)
  return out.reshape(b, t, table.shape[1])


# speedup vs baseline: 1.0001x; 1.0001x over previous
import functools

import jax
import jax.numpy as jnp
from jax import lax
from jax.experimental import pallas as pl
from jax.experimental.pallas import tpu as pltpu
from jax.experimental.pallas import tpu_sc as plsc

_NUM_CORES = 2
_NUM_SUBCORES = 16
_NUM_WORKERS = _NUM_CORES * _NUM_SUBCORES

_CHUNK = 16
_NBUF = 7
_B_PER_W = 1024
_N_CHUNKS = _B_PER_W // _CHUNK


def _make_gather(n_rows: int, d_model: int):
  mesh = plsc.VectorSubcoreMesh(core_axis_name="c", subcore_axis_name="s")

  @functools.partial(
      pl.kernel,
      mesh=mesh,
      out_type=jax.ShapeDtypeStruct((n_rows, d_model), jnp.float32),
      scratch_types=[
          pltpu.VMEM((_N_CHUNKS, _CHUNK), jnp.int32),
          pltpu.VMEM((_NBUF, _CHUNK, d_model), jnp.float32),
          [pltpu.SemaphoreType.DMA] * _NBUF,
          [pltpu.SemaphoreType.DMA] * _NBUF,
      ],
  )
  def gather_kernel(idx_hbm, table_hbm, out_hbm, idx_v, rows_v, gsems, wsems):
    wid = lax.axis_index("s") * _NUM_CORES + lax.axis_index("c")
    base = wid * _B_PER_W
    pltpu.sync_copy(idx_hbm.at[wid], idx_v)

    def g_copy(c, b):
      return pltpu.make_async_copy(
          table_hbm.at[idx_v.at[c]], rows_v.at[b], gsems[b])

    def w_copy(c, b):
      dst = out_hbm.at[pl.ds(base + c * _CHUNK, _CHUNK)]
      return pltpu.make_async_copy(rows_v.at[b], dst, wsems[b])

    for c in range(_NBUF - 1):
      g_copy(c, c).start()

    _MAIN = _N_CHUNKS - (_N_CHUNKS % _NBUF)

    @pl.loop(0, _MAIN, step=_NBUF)
    def _chunks(c0):
      for b in range(_NBUF):
        c = c0 + b
        g_copy(c, b).wait()
        w_copy(c, b).start()
        b2 = (b + _NBUF - 1) % _NBUF

        @pl.when(c > 0)
        def _():
          w_copy(c - 1, b2).wait()

        @pl.when(c + _NBUF - 1 < _N_CHUNKS)
        def _():
          g_copy(c + _NBUF - 1, b2).start()

    for c in range(_MAIN, _N_CHUNKS):
      b = c % _NBUF
      g_copy(c, b).wait()
      w_copy(c, b).start()
      w_copy(c - 1, (b + _NBUF - 1) % _NBUF).wait()

    w_copy(_N_CHUNKS - 1, (_N_CHUNKS - 1) % _NBUF).wait()

  return gather_kernel


def kernel(positions, table):
  b, t = positions.shape
  n = b * t
  idx = positions.reshape(_NUM_WORKERS, _N_CHUNKS, _CHUNK).astype(jnp.int32)
  out = _make_gather(n, table.shape[1])(idx, table)
  return out.reshape(b, t, table.shape[1])


# speedup vs baseline: 1.0052x; 1.0051x over previous
import functools

import jax
import jax.numpy as jnp
from jax import lax
from jax.experimental import pallas as pl
from jax.experimental.pallas import tpu as pltpu
from jax.experimental.pallas import tpu_sc as plsc

_NUM_CORES = 2
_NUM_SUBCORES = 16
_NUM_WORKERS = _NUM_CORES * _NUM_SUBCORES

_CHUNK = 16
_NBUF = 6
_B_PER_W = 1024
_N_CHUNKS = _B_PER_W // _CHUNK


def _make_gather(n_rows: int, d_model: int):
  mesh = plsc.VectorSubcoreMesh(core_axis_name="c", subcore_axis_name="s")

  @functools.partial(
      pl.kernel,
      mesh=mesh,
      out_type=jax.ShapeDtypeStruct((n_rows, d_model), jnp.float32),
      scratch_types=[
          pltpu.VMEM((_N_CHUNKS, _CHUNK), jnp.int32),
          pltpu.VMEM((_NBUF, _CHUNK, d_model), jnp.float32),
          [pltpu.SemaphoreType.DMA] * _NBUF,
          [pltpu.SemaphoreType.DMA] * _NBUF,
      ],
  )
  def gather_kernel(idx_hbm, table_hbm, out_hbm, idx_v, rows_v, gsems, wsems):
    wid = lax.axis_index("s") * _NUM_CORES + lax.axis_index("c")
    base = wid * _B_PER_W
    pltpu.sync_copy(idx_hbm.at[wid], idx_v)

    def g_copy(c, b):
      return pltpu.make_async_copy(
          table_hbm.at[idx_v.at[c]], rows_v.at[b], gsems[b])

    def w_copy(c, b):
      dst = out_hbm.at[pl.ds(base + c * _CHUNK, _CHUNK)]
      return pltpu.make_async_copy(rows_v.at[b], dst, wsems[b])

    for c in range(_NBUF - 1):
      g_copy(c, c).start()

    _MAIN = _N_CHUNKS - (_N_CHUNKS % _NBUF)

    @pl.loop(0, _MAIN, step=_NBUF)
    def _chunks(c0):
      for b in range(_NBUF):
        c = c0 + b
        g_copy(c, b).wait()
        w_copy(c, b).start()
        b2 = (b + _NBUF - 1) % _NBUF

        @pl.when(c > 0)
        def _():
          w_copy(c - 1, b2).wait()

        @pl.when(c + _NBUF - 1 < _N_CHUNKS)
        def _():
          g_copy(c + _NBUF - 1, b2).start()

    for c in range(_MAIN, _N_CHUNKS):
      b = c % _NBUF
      g_copy(c, b).wait()
      w_copy(c, b).start()
      w_copy(c - 1, (b + _NBUF - 1) % _NBUF).wait()

    w_copy(_N_CHUNKS - 1, (_N_CHUNKS - 1) % _NBUF).wait()

  return gather_kernel


def kernel(positions, table):
  b, t = positions.shape
  n = b * t
  idx = positions.reshape(_NUM_WORKERS, _N_CHUNKS, _CHUNK).astype(jnp.int32)
  out = _make_gather(n, table.shape[1])(idx, table)
  return out.reshape(b, t, table.shape[1])
